# baseline (speedup 1.0000x reference)
import functools

import jax
import jax.numpy as jnp
from jax import lax
from jax.experimental import pallas as pl
from jax.experimental.pallas import tpu as pltpu
from jax.experimental.pallas import tpu_sc as plsc

_NUM_CORES = 2
_NUM_SUBCORES = 16
_NW = _NUM_CORES * _NUM_SUBCORES
_LANES = 16
_REPLICAS = 8


def _make_gather(V, D, B, R):
    assert B % (8 * _NW) == 0
    b_per_w = B // _NW
    mesh = plsc.VectorSubcoreMesh(core_axis_name="c", subcore_axis_name="s")

    n_chunks = 2
    rows_c = b_per_w // n_chunks

    @functools.partial(
        pl.kernel,
        mesh=mesh,
        out_type=jax.ShapeDtypeStruct((B, D), jnp.float32),
        scratch_types=[
            pltpu.VMEM((b_per_w,), jnp.int32),
            [pltpu.VMEM((rows_c, D), jnp.float32) for _ in range(n_chunks)],
            [pltpu.SemaphoreType.DMA for _ in range(n_chunks)],
            [pltpu.SemaphoreType.DMA for _ in range(n_chunks)],
        ],
    )
    def k(table_hbm, idx_hbm, out_hbm, idx_v, bufs, gsems, ssems):
        wid = lax.axis_index("s") * _NUM_CORES + lax.axis_index("c")
        base = wid * b_per_w
        pltpu.sync_copy(idx_hbm.at[pl.ds(base, b_per_w)], idx_v)
        off = (wid % R) * V
        for i in range(b_per_w // _LANES):
            sl = pl.ds(i * _LANES, _LANES)
            idx_v[sl] = idx_v[sl] + off
        scat = []
        for j in range(n_chunks):
            pltpu.async_copy(
                table_hbm.at[idx_v.at[pl.ds(j * rows_c, rows_c)]],
                bufs[j],
                gsems[j],
            ).wait()
            scat.append(
                pltpu.async_copy(
                    bufs[j], out_hbm.at[pl.ds(base + j * rows_c, rows_c)], ssems[j]
                )
            )
        for s in scat:
            s.wait()

    return k


def kernel(obj_labels, obj_embedding_weight):
    B = obj_labels.shape[0]
    V, D = obj_embedding_weight.shape
    table_rep = jnp.tile(obj_embedding_weight, (_REPLICAS, 1))
    return _make_gather(V, D, B, _REPLICAS)(table_rep, obj_labels)


# speedup vs baseline: 1.0436x; 1.0436x over previous
import functools

import jax
import jax.numpy as jnp
from jax import lax
from jax.experimental import pallas as pl
from jax.experimental.pallas import tpu as pltpu
from jax.experimental.pallas import tpu_sc as plsc

_NUM_CORES = 2
_NUM_SUBCORES = 16
_NW = _NUM_CORES * _NUM_SUBCORES
_LANES = 16
_REPLICAS = 8


def _make_gather(V, D, B, R):
    assert B % (8 * _NW) == 0
    b_per_w = B // _NW
    mesh = plsc.VectorSubcoreMesh(core_axis_name="c", subcore_axis_name="s")

    @functools.partial(
        pl.kernel,
        mesh=mesh,
        out_type=jax.ShapeDtypeStruct((B, D), jnp.float32),
        scratch_types=[
            pltpu.VMEM((b_per_w,), jnp.int32),
            pltpu.VMEM((b_per_w, D), jnp.float32),
            pltpu.SemaphoreType.DMA,
        ],
    )
    def k(table_hbm, idx_hbm, out_hbm, idx_v, rows_v, sem):
        wid = lax.axis_index("s") * _NUM_CORES + lax.axis_index("c")
        base = wid * b_per_w
        with jax.named_scope("idx_load"):
            pltpu.sync_copy(idx_hbm.at[pl.ds(base, b_per_w)], idx_v)
        with jax.named_scope("idx_off"):
            off = (wid % R) * V
            for i in range(b_per_w // _LANES):
                sl = pl.ds(i * _LANES, _LANES)
                idx_v[sl] = idx_v[sl] + off
        with jax.named_scope("row_gather"):
            pltpu.async_copy(table_hbm.at[idx_v], rows_v, sem).wait()
        with jax.named_scope("row_store"):
            pltpu.sync_copy(rows_v, out_hbm.at[pl.ds(base, b_per_w)])

    return k


def kernel(obj_labels, obj_embedding_weight):
    B = obj_labels.shape[0]
    V, D = obj_embedding_weight.shape
    table_rep = jnp.tile(obj_embedding_weight, (_REPLICAS, 1))
    return _make_gather(V, D, B, _REPLICAS)(table_rep, obj_labels)


# speedup vs baseline: 1.0674x; 1.0229x over previous
import functools

import jax
import jax.numpy as jnp
from jax import lax
from jax.experimental import pallas as pl
from jax.experimental.pallas import tpu as pltpu
from jax.experimental.pallas import tpu_sc as plsc

_NUM_CORES = 2
_NUM_SUBCORES = 16
_NW = _NUM_CORES * _NUM_SUBCORES


def _make_gather(V, D, B):
    assert B % (8 * _NW) == 0
    b_per_w = B // _NW
    rows_stage = 128
    n_full = V // rows_stage
    rows_rem = V - n_full * rows_stage
    mesh = plsc.VectorSubcoreMesh(core_axis_name="c", subcore_axis_name="s")

    @functools.partial(
        pl.kernel,
        mesh=mesh,
        out_type=jax.ShapeDtypeStruct((B, D), jnp.float32),
        scratch_types=[
            pltpu.VMEM_SHARED((V, D), jnp.float32),
            pltpu.VMEM((b_per_w,), jnp.int32),
            pltpu.VMEM((b_per_w, D), jnp.float32),
            pltpu.SemaphoreType.DMA,
        ],
    )
    def k(table_hbm, idx_hbm, out_hbm, table_sp, idx_v, rows_v, sem):
        cid = lax.axis_index("c")
        sid = lax.axis_index("s")
        wid = sid * _NUM_CORES + cid
        base = wid * b_per_w
        with jax.named_scope("table_stage"):
            @pl.when(sid < n_full)
            def _():
                r0 = pl.multiple_of(sid * rows_stage, 8)
                pltpu.sync_copy(
                    table_hbm.at[pl.ds(r0, rows_stage)],
                    table_sp.at[pl.ds(r0, rows_stage)],
                )

            if rows_rem:
                @pl.when(sid == n_full)
                def _():
                    pltpu.sync_copy(
                        table_hbm.at[pl.ds(n_full * rows_stage, rows_rem)],
                        table_sp.at[pl.ds(n_full * rows_stage, rows_rem)],
                    )
        with jax.named_scope("idx_load"):
            pltpu.sync_copy(idx_hbm.at[pl.ds(base, b_per_w)], idx_v)
        plsc.subcore_barrier()
        with jax.named_scope("row_gather"):
            pltpu.async_copy(table_sp.at[idx_v], rows_v, sem).wait()
        with jax.named_scope("row_store"):
            pltpu.sync_copy(rows_v, out_hbm.at[pl.ds(base, b_per_w)])

    return k


def kernel(obj_labels, obj_embedding_weight):
    B = obj_labels.shape[0]
    V, D = obj_embedding_weight.shape
    return _make_gather(V, D, B)(obj_embedding_weight, obj_labels)


# speedup vs baseline: 1.0947x; 1.0255x over previous
import functools

import jax
import jax.numpy as jnp
from jax import lax
from jax.experimental import pallas as pl
from jax.experimental.pallas import tpu as pltpu
from jax.experimental.pallas import tpu_sc as plsc

_NUM_CORES = 2
_NUM_SUBCORES = 16
_NW = _NUM_CORES * _NUM_SUBCORES


def _make_gather(V, D, B):
    assert B % (8 * _NW) == 0
    b_per_w = B // _NW
    rows_stage = 64
    n_full = V // rows_stage
    rows_rem = V - n_full * rows_stage
    mesh = plsc.VectorSubcoreMesh(core_axis_name="c", subcore_axis_name="s")

    @functools.partial(
        pl.kernel,
        mesh=mesh,
        out_type=jax.ShapeDtypeStruct((B, D), jnp.float32),
        scratch_types=[
            pltpu.VMEM_SHARED((V, D), jnp.float32),
            pltpu.VMEM((b_per_w,), jnp.int32),
            pltpu.VMEM((b_per_w, D), jnp.float32),
            pltpu.SemaphoreType.DMA,
            pltpu.SemaphoreType.DMA,
        ],
    )
    def k(table_hbm, idx_hbm, out_hbm, table_sp, idx_v, rows_v, gsem, isem):
        cid = lax.axis_index("c")
        sid = lax.axis_index("s")
        wid = sid * _NUM_CORES + cid
        base = wid * b_per_w
        ih = pltpu.async_copy(idx_hbm.at[pl.ds(base, b_per_w)], idx_v, isem)

        @pl.when(sid < n_full)
        def _():
            r0 = pl.multiple_of(sid * rows_stage, 8)
            pltpu.sync_copy(
                table_hbm.at[pl.ds(r0, rows_stage)],
                table_sp.at[pl.ds(r0, rows_stage)],
            )

        if rows_rem:
            @pl.when(sid == n_full)
            def _():
                pltpu.sync_copy(
                    table_hbm.at[pl.ds(n_full * rows_stage, rows_rem)],
                    table_sp.at[pl.ds(n_full * rows_stage, rows_rem)],
                )

        plsc.subcore_barrier()
        ih.wait()
        pltpu.async_copy(table_sp.at[idx_v], rows_v, gsem).wait()
        pltpu.sync_copy(rows_v, out_hbm.at[pl.ds(base, b_per_w)])

    return k


def kernel(obj_labels, obj_embedding_weight):
    B = obj_labels.shape[0]
    V, D = obj_embedding_weight.shape
    return _make_gather(V, D, B)(obj_embedding_weight, obj_labels)
